# Initial kernel scaffold; baseline (speedup 1.0000x reference)
import jax
import jax.numpy as jnp
from jax.experimental import pallas as pl


def kernel(logits, beam_scores):
    raise NotImplementedError("write your pallas kernel here")


# speedup vs baseline: 95.1182x; 95.1182x over previous
import functools

import jax
import jax.numpy as jnp
from jax import lax
from jax.experimental import pallas as pl
from jax.experimental.pallas import tpu as pltpu
from jax.experimental.pallas import tpu_sc as plsc

NC, NS, L = 2, 16, 16
NW = NC * NS

_B, _K, _V = 64, 4, 100000
ROWS = _B * _K
RPW = ROWS // NW
BPW = RPW // _K
CH = 20000
NCH = _V // CH
BLKV = 5
NBLK = CH // (BLKV * L)
TOT = RPW * NCH


def _insert(vv, ib, st):
  m0, m1, m2, m3, i0, i1, i2, i3 = st
  gt = vv > m0
  nm0 = jnp.where(gt, vv, m0); ni0 = jnp.where(gt, ib, i0)
  cv = jnp.where(gt, m0, vv); ci = jnp.where(gt, i0, ib)
  gt = cv > m1
  nm1 = jnp.where(gt, cv, m1); ni1 = jnp.where(gt, ci, i1)
  cv = jnp.where(gt, m1, cv); ci = jnp.where(gt, i1, ci)
  gt = cv > m2
  nm2 = jnp.where(gt, cv, m2); ni2 = jnp.where(gt, ci, i2)
  cv = jnp.where(gt, m2, cv); ci = jnp.where(gt, i2, ci)
  gt = cv > m3
  nm3 = jnp.where(gt, cv, m3); ni3 = jnp.where(gt, ci, i3)
  return (nm0, nm1, nm2, nm3, ni0, ni1, ni2, ni3)


def _beam_body(logits_hbm, bs_hbm, syms_hbm, scores_hbm, from_hbm, to_hbm,
               buf, mv, mi, lanebuf, xsf, xtf, bsl,
               osym, osc, ofr, oto, sems):
  wid = lax.axis_index("s") * NC + lax.axis_index("c")
  row0 = wid * RPW
  b0 = wid * BPW

  iota = lax.iota(jnp.int32, L)
  depth = iota & 3
  quad = iota >> 2
  low4 = iota < 4
  negvec = jnp.full((L,), -jnp.inf, jnp.float32)
  zeroi = jnp.zeros((L,), jnp.int32)
  bigi = jnp.full((L,), jnp.int32(0x7FFFFFFF))

  pltpu.sync_copy(bs_hbm.at[pl.ds(b0, BPW)], bsl)

  def dma(g, slot):
    row = g // NCH
    c = g - row * NCH
    return pltpu.make_async_copy(
        logits_hbm.at[row0 + row, pl.ds(c * CH, CH)],
        buf.at[slot], sems.at[slot])

  dma(0, 0).start()

  def g_body(g, state):
    slot = lax.rem(g, 2)
    row = g // NCH
    c = g - row * NCH

    @pl.when(g + 1 < TOT)
    def _():
      dma(g + 1, 1 - slot).start()

    dma(g, slot).wait()

    freshm = jnp.broadcast_to(c, (L,)) == 0
    m = [jnp.where(freshm, negvec, state[t]) for t in range(4)]
    ii = [jnp.where(freshm, zeroi, state[4 + t]) for t in range(4)]
    state = (*m, *ii)

    def blk_body(k, st):
      base = k * (BLKV * L)
      vs = [buf[slot, pl.ds(base + j * L, L)] for j in range(BLKV)]
      mx = vs[0]
      for j in range(1, BLKV):
        mx = jnp.maximum(mx, vs[j])
      pred = jnp.any(mx > st[3])

      def do_ins(s):
        pos0 = c * CH + base
        for j in range(BLKV):
          s = _insert(vs[j], iota + (pos0 + j * L), s)
        return s

      return lax.cond(pred, do_ins, lambda s: s, st)

    state = lax.fori_loop(0, NBLK, blk_body, state)

    @pl.when(c == NCH - 1)
    def _():
      m0, m1, m2, m3, i0, i1, i2, i3 = state
      mv[0] = m0; mv[1] = m1; mv[2] = m2; mv[3] = m3
      mi[0] = i0; mi[1] = i1; mi[2] = i2; mi[3] = i3
      lv = m0
      lane_sel = zeroi
      for r in range(4):
        mval = jnp.max(lv)
        elig = lv == mval
        imin = jnp.min(jnp.where(elig, i0, bigi))
        hit = elig & (i0 == imin)
        lane = jnp.min(jnp.where(hit, iota, bigi))
        lane_sel = jnp.where(iota == r, lane, lane_sel)
        lv = jnp.where(hit, negvec, lv)
      lanebuf[...] = lane_sel
      lane4 = plsc.load_gather(lanebuf, [quad])
      cv = plsc.load_gather(mv, [depth, lane4])
      ci = plsc.load_gather(mi, [depth, lane4])
      sel_v = negvec
      sel_i = zeroi
      for r in range(4):
        mval = jnp.max(cv)
        elig = cv == mval
        imin = jnp.min(jnp.where(elig, ci, bigi))
        sel_v = jnp.where(iota == r, mval, sel_v)
        sel_i = jnp.where(iota == r, imin, sel_i)
        cv = jnp.where(elig & (ci == imin), negvec, cv)
      bl = row >> 2
      j = row & 3
      blv = jnp.broadcast_to(bl, (L,))
      dst = depth + j * 4
      plsc.store_scatter(xsf, [blv, dst], sel_v, mask=low4)
      plsc.store_scatter(xtf, [blv, dst], sel_i, mask=low4)

    return state

  lax.fori_loop(0, TOT, g_body, (negvec,) * 4 + (zeroi,) * 4)

  for bl in range(BPW):
    blv = jnp.full((L,), bl, jnp.int32)
    xs = xsf[bl]
    bs_g = plsc.load_gather(bsl, [blv, quad])
    cv = bs_g + xs
    tk = negvec
    tv = zeroi
    for r in range(4):
      mval = jnp.max(cv)
      elig = cv == mval
      imin = jnp.min(jnp.where(elig, iota, bigi))
      tk = jnp.where(iota == r, mval, tk)
      tv = jnp.where(iota == r, imin, tv)
      cv = jnp.where(elig & (iota == imin), negvec, cv)
    fr = tv >> 2
    to = tv & 3
    sym = plsc.load_gather(xtf, [blv, tv])
    plsc.store_scatter(osym, [blv, depth], sym, mask=low4)
    plsc.store_scatter(osc, [blv, depth], tk, mask=low4)
    plsc.store_scatter(ofr, [blv, depth], fr, mask=low4)
    plsc.store_scatter(oto, [blv, depth], to, mask=low4)

  pltpu.sync_copy(osym, syms_hbm.at[pl.ds(b0, BPW)])
  pltpu.sync_copy(osc, scores_hbm.at[pl.ds(b0, BPW)])
  pltpu.sync_copy(ofr, from_hbm.at[pl.ds(b0, BPW)])
  pltpu.sync_copy(oto, to_hbm.at[pl.ds(b0, BPW)])


_beam_call = functools.partial(
    pl.kernel,
    out_type=(
        jax.ShapeDtypeStruct((_B, _K), jnp.int32),
        jax.ShapeDtypeStruct((_B, _K), jnp.float32),
        jax.ShapeDtypeStruct((_B, _K), jnp.int32),
        jax.ShapeDtypeStruct((_B, _K), jnp.int32),
    ),
    mesh=plsc.VectorSubcoreMesh(core_axis_name="c", subcore_axis_name="s",
                                num_cores=NC, num_subcores=NS),
    compiler_params=pltpu.CompilerParams(use_tc_tiling_on_sc=False,
                                         needs_layout_passes=False),
    scratch_types=[
        pltpu.VMEM((2, CH), jnp.float32),
        pltpu.VMEM((_K, L), jnp.float32),
        pltpu.VMEM((_K, L), jnp.int32),
        pltpu.VMEM((L,), jnp.int32),
        pltpu.VMEM((BPW, L), jnp.float32),
        pltpu.VMEM((BPW, L), jnp.int32),
        pltpu.VMEM((BPW, _K), jnp.float32),
        pltpu.VMEM((BPW, _K), jnp.int32),
        pltpu.VMEM((BPW, _K), jnp.float32),
        pltpu.VMEM((BPW, _K), jnp.int32),
        pltpu.VMEM((BPW, _K), jnp.int32),
        pltpu.SemaphoreType.DMA((2,)),
    ],
)


@jax.jit
def kernel(logits, beam_scores):
  Bb, K, V = logits.shape
  logits2 = logits.reshape(Bb * K, V)
  syms, scores, fr, to = _beam_call(_beam_body)(logits2, beam_scores)
  return syms, scores, fr, to


# speedup vs baseline: 120.8365x; 1.2704x over previous
import functools

import jax
import jax.numpy as jnp
from jax import lax
from jax.experimental import pallas as pl
from jax.experimental.pallas import tpu as pltpu
from jax.experimental.pallas import tpu_sc as plsc

NC, NS, L = 2, 16, 16
NW = NC * NS

_B, _K, _V = 64, 4, 100000
ROWS = _B * _K
RPW = ROWS // NW
BPW = RPW // _K
CH = 20000
NCH = _V // CH
GRPV = 5
NGRP = 5
BLKV = GRPV * NGRP
NBLK = CH // (BLKV * L)
TOT = RPW * NCH


def _insert(vv, ib, st):
  m0, m1, m2, m3, i0, i1, i2, i3 = st
  gt = vv > m0
  nm0 = jnp.where(gt, vv, m0); ni0 = jnp.where(gt, ib, i0)
  cv = jnp.where(gt, m0, vv); ci = jnp.where(gt, i0, ib)
  gt = cv > m1
  nm1 = jnp.where(gt, cv, m1); ni1 = jnp.where(gt, ci, i1)
  cv = jnp.where(gt, m1, cv); ci = jnp.where(gt, i1, ci)
  gt = cv > m2
  nm2 = jnp.where(gt, cv, m2); ni2 = jnp.where(gt, ci, i2)
  cv = jnp.where(gt, m2, cv); ci = jnp.where(gt, i2, ci)
  gt = cv > m3
  nm3 = jnp.where(gt, cv, m3); ni3 = jnp.where(gt, ci, i3)
  return (nm0, nm1, nm2, nm3, ni0, ni1, ni2, ni3)


def _beam_body(logits_hbm, bs_hbm, syms_hbm, scores_hbm, from_hbm, to_hbm,
               buf, mv, mi, lanebuf, xsf, xtf, bsl,
               osym, osc, ofr, oto, sems):
  wid = lax.axis_index("s") * NC + lax.axis_index("c")
  row0 = wid * RPW
  b0 = wid * BPW

  iota = lax.iota(jnp.int32, L)
  depth = iota & 3
  quad = iota >> 2
  low4 = iota < 4
  negvec = jnp.full((L,), -jnp.inf, jnp.float32)
  zeroi = jnp.zeros((L,), jnp.int32)
  bigi = jnp.full((L,), jnp.int32(0x7FFFFFFF))

  pltpu.sync_copy(bs_hbm.at[pl.ds(b0, BPW)], bsl)

  def dma(g, slot):
    row = g // NCH
    c = g - row * NCH
    return pltpu.make_async_copy(
        logits_hbm.at[row0 + row, pl.ds(c * CH, CH)],
        buf.at[slot], sems.at[slot])

  dma(0, 0).start()

  def g_body(g, state):
    slot = lax.rem(g, 2)
    row = g // NCH
    c = g - row * NCH

    @pl.when(g + 1 < TOT)
    def _():
      dma(g + 1, 1 - slot).start()

    dma(g, slot).wait()

    freshm = jnp.broadcast_to(c, (L,)) == 0
    m = [jnp.where(freshm, negvec, state[t]) for t in range(4)]
    ii = [jnp.where(freshm, zeroi, state[4 + t]) for t in range(4)]
    state = (*m, *ii)

    def blk_body(k, st):
      base = k * (BLKV * L)
      vs = [buf[slot, pl.ds(base + j * L, L)] for j in range(BLKV)]
      gmx = []
      for gi in range(NGRP):
        a = vs[GRPV * gi]
        for j in range(1, GRPV):
          a = jnp.maximum(a, vs[GRPV * gi + j])
        gmx.append(a)
      mx = jnp.maximum(jnp.maximum(gmx[0], gmx[1]),
                       jnp.maximum(jnp.maximum(gmx[2], gmx[3]), gmx[4]))
      pred = jnp.any(mx > st[3])

      def do_blk(s):
        pos0 = c * CH + base
        for gi in range(NGRP):
          sub = jnp.any(gmx[gi] > s[3])

          def do_sub(ss, gi=gi):
            for j in range(GRPV):
              q = GRPV * gi + j
              ss = _insert(vs[q], iota + (pos0 + q * L), ss)
            return ss

          s = lax.cond(sub, do_sub, lambda ss: ss, s)
        return s

      return lax.cond(pred, do_blk, lambda s: s, st)

    state = lax.fori_loop(0, NBLK, blk_body, state)

    @pl.when(c == NCH - 1)
    def _():
      m0, m1, m2, m3, i0, i1, i2, i3 = state
      mv[0] = m0; mv[1] = m1; mv[2] = m2; mv[3] = m3
      mi[0] = i0; mi[1] = i1; mi[2] = i2; mi[3] = i3
      lv = m0
      lane_sel = zeroi
      for r in range(4):
        mval = jnp.max(lv)
        elig = lv == mval
        imin = jnp.min(jnp.where(elig, i0, bigi))
        hit = elig & (i0 == imin)
        lane = jnp.min(jnp.where(hit, iota, bigi))
        lane_sel = jnp.where(iota == r, lane, lane_sel)
        lv = jnp.where(hit, negvec, lv)
      lanebuf[...] = lane_sel
      lane4 = plsc.load_gather(lanebuf, [quad])
      cv = plsc.load_gather(mv, [depth, lane4])
      ci = plsc.load_gather(mi, [depth, lane4])
      sel_v = negvec
      sel_i = zeroi
      for r in range(4):
        mval = jnp.max(cv)
        elig = cv == mval
        imin = jnp.min(jnp.where(elig, ci, bigi))
        sel_v = jnp.where(iota == r, mval, sel_v)
        sel_i = jnp.where(iota == r, imin, sel_i)
        cv = jnp.where(elig & (ci == imin), negvec, cv)
      bl = row >> 2
      j = row & 3
      blv = jnp.broadcast_to(bl, (L,))
      dst = depth + j * 4
      plsc.store_scatter(xsf, [blv, dst], sel_v, mask=low4)
      plsc.store_scatter(xtf, [blv, dst], sel_i, mask=low4)

    return state

  lax.fori_loop(0, TOT, g_body, (negvec,) * 4 + (zeroi,) * 4)

  for bl in range(BPW):
    blv = jnp.full((L,), bl, jnp.int32)
    xs = xsf[bl]
    bs_g = plsc.load_gather(bsl, [blv, quad])
    cv = bs_g + xs
    tk = negvec
    tv = zeroi
    for r in range(4):
      mval = jnp.max(cv)
      elig = cv == mval
      imin = jnp.min(jnp.where(elig, iota, bigi))
      tk = jnp.where(iota == r, mval, tk)
      tv = jnp.where(iota == r, imin, tv)
      cv = jnp.where(elig & (iota == imin), negvec, cv)
    fr = tv >> 2
    to = tv & 3
    sym = plsc.load_gather(xtf, [blv, tv])
    plsc.store_scatter(osym, [blv, depth], sym, mask=low4)
    plsc.store_scatter(osc, [blv, depth], tk, mask=low4)
    plsc.store_scatter(ofr, [blv, depth], fr, mask=low4)
    plsc.store_scatter(oto, [blv, depth], to, mask=low4)

  pltpu.sync_copy(osym, syms_hbm.at[pl.ds(b0, BPW)])
  pltpu.sync_copy(osc, scores_hbm.at[pl.ds(b0, BPW)])
  pltpu.sync_copy(ofr, from_hbm.at[pl.ds(b0, BPW)])
  pltpu.sync_copy(oto, to_hbm.at[pl.ds(b0, BPW)])


_beam_call = functools.partial(
    pl.kernel,
    out_type=(
        jax.ShapeDtypeStruct((_B, _K), jnp.int32),
        jax.ShapeDtypeStruct((_B, _K), jnp.float32),
        jax.ShapeDtypeStruct((_B, _K), jnp.int32),
        jax.ShapeDtypeStruct((_B, _K), jnp.int32),
    ),
    mesh=plsc.VectorSubcoreMesh(core_axis_name="c", subcore_axis_name="s",
                                num_cores=NC, num_subcores=NS),
    compiler_params=pltpu.CompilerParams(use_tc_tiling_on_sc=False,
                                         needs_layout_passes=False),
    scratch_types=[
        pltpu.VMEM((2, CH), jnp.float32),
        pltpu.VMEM((_K, L), jnp.float32),
        pltpu.VMEM((_K, L), jnp.int32),
        pltpu.VMEM((L,), jnp.int32),
        pltpu.VMEM((BPW, L), jnp.float32),
        pltpu.VMEM((BPW, L), jnp.int32),
        pltpu.VMEM((BPW, _K), jnp.float32),
        pltpu.VMEM((BPW, _K), jnp.int32),
        pltpu.VMEM((BPW, _K), jnp.float32),
        pltpu.VMEM((BPW, _K), jnp.int32),
        pltpu.VMEM((BPW, _K), jnp.int32),
        pltpu.SemaphoreType.DMA((2,)),
    ],
)


@jax.jit
def kernel(logits, beam_scores):
  Bb, K, V = logits.shape
  logits2 = logits.reshape(Bb * K, V)
  syms, scores, fr, to = _beam_call(_beam_body)(logits2, beam_scores)
  return syms, scores, fr, to
